# baseline (speedup 1.0000x reference)
import jax
import jax.numpy as jnp
from jax.experimental import pallas as pl
from jax.experimental.pallas import tpu as pltpu

_T = 0.05
_QTILE = 128
_KTILE = 16384


def _mm_kernel(x_ref, m_ref, o_ref):
    o_ref[...] = jax.lax.dot_general(
        x_ref[...] / _T,
        m_ref[...],
        dimension_numbers=(((1,), (1,)), ((), ())),
        preferred_element_type=jnp.float32,
    )


def kernel(x, mem_feat):
    q, d = x.shape
    k = mem_feat.shape[0]
    grid = (k // _KTILE, q // _QTILE)
    return pl.pallas_call(
        _mm_kernel,
        grid=grid,
        in_specs=[
            pl.BlockSpec((_QTILE, d), lambda i, j: (j, 0)),
            pl.BlockSpec((_KTILE, d), lambda i, j: (i, 0)),
        ],
        out_specs=pl.BlockSpec((_QTILE, _KTILE), lambda i, j: (j, i)),
        out_shape=jax.ShapeDtypeStruct((q, k), jnp.float32),
        compiler_params=pltpu.CompilerParams(
            dimension_semantics=("parallel", "parallel"),
        ),
    )(x, mem_feat)


# speedup vs baseline: 1.3100x; 1.3100x over previous
import jax
import jax.numpy as jnp
from jax.experimental import pallas as pl
from jax.experimental.pallas import tpu as pltpu

_T = 0.05
_KTILE = 4096
_NSPLIT = 4
_Q = 1024
_RS = _Q // _NSPLIT


def _wait_slot(buf, o_ref, sems, slot):
    for s in range(_NSPLIT):
        pltpu.make_async_copy(
            buf.at[pl.ds(s * _RS, _RS)],
            o_ref.at[pl.ds(s * _RS, _RS), pl.ds(0, _KTILE)],
            sems.at[slot, s],
        ).wait()


def _start_slot(buf, o_ref, sems, slot, col):
    for s in range(_NSPLIT):
        pltpu.make_async_copy(
            buf.at[pl.ds(s * _RS, _RS)],
            o_ref.at[pl.ds(s * _RS, _RS), pl.ds(col, _KTILE)],
            sems.at[slot, s],
        ).start()


def _mm_kernel(x_ref, m_ref, o_ref, s0, s1, sems):
    i = pl.program_id(0)
    n = pl.num_programs(0)
    acc = jax.lax.dot_general(
        x_ref[...] / _T,
        m_ref[...],
        dimension_numbers=(((1,), (1,)), ((), ())),
        preferred_element_type=jnp.float32,
    )
    col = i * _KTILE

    @pl.when(i % 2 == 0)
    def _():
        @pl.when(i >= 2)
        def _():
            _wait_slot(s0, o_ref, sems, 0)

        s0[...] = acc
        _start_slot(s0, o_ref, sems, 0, col)

    @pl.when(i % 2 == 1)
    def _():
        @pl.when(i >= 2)
        def _():
            _wait_slot(s1, o_ref, sems, 1)

        s1[...] = acc
        _start_slot(s1, o_ref, sems, 1, col)

    @pl.when(i == n - 1)
    def _():
        _wait_slot(s0, o_ref, sems, 0)
        _wait_slot(s1, o_ref, sems, 1)


def kernel(x, mem_feat):
    q, d = x.shape
    k = mem_feat.shape[0]
    grid = (k // _KTILE,)
    return pl.pallas_call(
        _mm_kernel,
        grid=grid,
        in_specs=[
            pl.BlockSpec((q, d), lambda i: (0, 0)),
            pl.BlockSpec((_KTILE, d), lambda i: (i, 0)),
        ],
        out_specs=pl.BlockSpec(memory_space=pltpu.HBM),
        out_shape=jax.ShapeDtypeStruct((q, k), jnp.float32),
        scratch_shapes=[
            pltpu.VMEM((q, _KTILE), jnp.float32),
            pltpu.VMEM((q, _KTILE), jnp.float32),
            pltpu.SemaphoreType.DMA((2, _NSPLIT)),
        ],
        compiler_params=pltpu.CompilerParams(
            dimension_semantics=("arbitrary",),
        ),
    )(x, mem_feat)


# speedup vs baseline: 1.3295x; 1.0149x over previous
import jax
import jax.numpy as jnp
from jax.experimental import pallas as pl
from jax.experimental.pallas import tpu as pltpu

_T = 0.05
_KTILE = 4096


def _mm_kernel(x_ref, m_ref, o_ref):
    o_ref[...] = jax.lax.dot_general(
        x_ref[...] / _T,
        m_ref[...],
        dimension_numbers=(((1,), (1,)), ((), ())),
        preferred_element_type=jnp.float32,
    )


def kernel(x, mem_feat):
    q, d = x.shape
    k = mem_feat.shape[0]
    grid = (k // _KTILE,)
    return pl.pallas_call(
        _mm_kernel,
        grid=grid,
        in_specs=[
            pl.BlockSpec((q, d), lambda i: (0, 0)),
            pl.BlockSpec((_KTILE, d), lambda i: (i, 0)),
        ],
        out_specs=pl.BlockSpec((q, _KTILE), lambda i: (0, i)),
        out_shape=jax.ShapeDtypeStruct((q, k), jnp.float32),
        compiler_params=pltpu.CompilerParams(
            dimension_semantics=("parallel",),
        ),
    )(x, mem_feat)
